# baseline (speedup 1.0000x reference)
import functools

import jax
import jax.numpy as jnp
from jax import lax
from jax.experimental import pallas as pl
from jax.experimental.pallas import tpu as pltpu
from jax.experimental.pallas import tpu_sc as plsc

_CHUNK = 128


@functools.lru_cache(maxsize=None)
def _build(batch: int, seq_len: int, hidden: int):
    info = plsc.get_sparse_core_info()
    num_workers = info.num_cores * info.num_subcores
    total_rows = batch * seq_len
    rows_per_worker = total_rows // num_workers
    n_chunks = rows_per_worker // _CHUNK
    assert rows_per_worker * num_workers == total_rows
    assert n_chunks * _CHUNK == rows_per_worker
    assert seq_len % rows_per_worker == 0

    mesh = plsc.VectorSubcoreMesh(core_axis_name="c", subcore_axis_name="s")

    @functools.partial(
        pl.kernel,
        out_type=jax.ShapeDtypeStruct((batch, seq_len, hidden), jnp.float32),
        mesh=mesh,
        scratch_types=[
            pltpu.VMEM((n_chunks, _CHUNK), jnp.int32),
            pltpu.VMEM((rows_per_worker, hidden), jnp.float32),
            pltpu.SemaphoreType.DMA,
        ],
    )
    def emb_kernel(ids_hbm, table_hbm, pos_hbm, out_hbm, idx_v, rows_v, sem):
        wid = lax.axis_index("s") * info.num_cores + lax.axis_index("c")
        base = wid * rows_per_worker
        b = lax.div(base, seq_len)
        pos_base = lax.rem(base, seq_len)
        for j in range(n_chunks):
            pltpu.sync_copy(ids_hbm.at[b, pl.ds(pos_base + j * _CHUNK, _CHUNK)],
                            idx_v.at[j])
        pltpu.sync_copy(pos_hbm.at[pl.ds(pos_base, rows_per_worker)], rows_v)
        copies = [
            pltpu.async_copy(
                table_hbm.at[idx_v.at[j]],
                rows_v.at[pl.ds(j * _CHUNK, _CHUNK)],
                sem,
                add=True,
            )
            for j in range(n_chunks)
        ]
        for cp in copies:
            cp.wait()
        pltpu.sync_copy(rows_v, out_hbm.at[b, pl.ds(pos_base, rows_per_worker)])

    return emb_kernel


def kernel(input_ids, embedding, position_embedding):
    batch, seq_len = input_ids.shape
    hidden = embedding.shape[1]
    fn = _build(batch, seq_len, hidden)
    return fn(input_ids.astype(jnp.int32), embedding, position_embedding)


# speedup vs baseline: 1.0283x; 1.0283x over previous
import functools

import jax
import jax.numpy as jnp
from jax import lax
from jax.experimental import pallas as pl
from jax.experimental.pallas import tpu as pltpu
from jax.experimental.pallas import tpu_sc as plsc

_CHUNK = 128


@functools.lru_cache(maxsize=None)
def _build(batch: int, seq_len: int, hidden: int):
    info = plsc.get_sparse_core_info()
    num_workers = info.num_cores * info.num_subcores
    total_rows = batch * seq_len
    rows_per_worker = total_rows // num_workers
    n_chunks = rows_per_worker // _CHUNK
    assert rows_per_worker * num_workers == total_rows
    assert n_chunks * _CHUNK == rows_per_worker
    assert seq_len % rows_per_worker == 0

    mesh = plsc.VectorSubcoreMesh(core_axis_name="c", subcore_axis_name="s")

    @functools.partial(
        pl.kernel,
        out_type=jax.ShapeDtypeStruct((batch, seq_len, hidden), jnp.float32),
        mesh=mesh,
        scratch_types=[
            pltpu.VMEM((rows_per_worker,), jnp.int32),
            pltpu.VMEM((rows_per_worker, hidden), jnp.float32),
            pltpu.SemaphoreType.DMA,
            pltpu.SemaphoreType.DMA,
            [pltpu.SemaphoreType.DMA for _ in range(n_chunks)],
            pltpu.SemaphoreType.DMA,
        ],
    )
    def emb_kernel(ids_hbm, table_hbm, pos_hbm, out_hbm,
                   idx_v, rows_v, sem_i, sem_p, sem_g, sem_o):
        wid = lax.axis_index("s") * info.num_cores + lax.axis_index("c")
        base = wid * rows_per_worker
        b = lax.div(base, seq_len)
        pos_base = lax.rem(base, seq_len)
        cp_i = pltpu.async_copy(
            ids_hbm.at[b, pl.ds(pos_base, rows_per_worker)], idx_v, sem_i)
        cp_p = pltpu.async_copy(
            pos_hbm.at[pl.ds(pos_base, rows_per_worker)], rows_v, sem_p)
        cp_i.wait()
        cp_p.wait()
        gathers = [
            pltpu.async_copy(
                table_hbm.at[idx_v.at[pl.ds(j * _CHUNK, _CHUNK)]],
                rows_v.at[pl.ds(j * _CHUNK, _CHUNK)],
                sem_g[j],
                add=True,
            )
            for j in range(n_chunks)
        ]
        outs = []
        for j in range(n_chunks):
            gathers[j].wait()
            outs.append(pltpu.async_copy(
                rows_v.at[pl.ds(j * _CHUNK, _CHUNK)],
                out_hbm.at[b, pl.ds(pos_base + j * _CHUNK, _CHUNK)],
                sem_o,
            ))
        for cp in outs:
            cp.wait()

    return emb_kernel


def kernel(input_ids, embedding, position_embedding):
    batch, seq_len = input_ids.shape
    hidden = embedding.shape[1]
    fn = _build(batch, seq_len, hidden)
    return fn(input_ids.astype(jnp.int32), embedding, position_embedding)


# speedup vs baseline: 1.0348x; 1.0064x over previous
import functools

import jax
import jax.numpy as jnp
from jax import lax
from jax.experimental import pallas as pl
from jax.experimental.pallas import tpu as pltpu
from jax.experimental.pallas import tpu_sc as plsc

_CHUNK = 64


@functools.lru_cache(maxsize=None)
def _build(batch: int, seq_len: int, hidden: int):
    info = plsc.get_sparse_core_info()
    num_workers = info.num_cores * info.num_subcores
    total_rows = batch * seq_len
    rows_per_worker = total_rows // num_workers
    n_chunks = rows_per_worker // _CHUNK
    assert rows_per_worker * num_workers == total_rows
    assert n_chunks * _CHUNK == rows_per_worker
    assert seq_len % rows_per_worker == 0

    mesh = plsc.VectorSubcoreMesh(core_axis_name="c", subcore_axis_name="s")

    @functools.partial(
        pl.kernel,
        out_type=jax.ShapeDtypeStruct((batch, seq_len, hidden), jnp.float32),
        mesh=mesh,
        scratch_types=[
            pltpu.VMEM((rows_per_worker,), jnp.int32),
            pltpu.VMEM((rows_per_worker, hidden), jnp.float32),
            pltpu.SemaphoreType.DMA,
            [pltpu.SemaphoreType.DMA for _ in range(n_chunks)],
            [pltpu.SemaphoreType.DMA for _ in range(n_chunks)],
            pltpu.SemaphoreType.DMA,
        ],
    )
    def emb_kernel(ids_hbm, table_hbm, pos_hbm, out_hbm,
                   idx_v, rows_v, sem_i, sem_p, sem_g, sem_o):
        wid = lax.axis_index("s") * info.num_cores + lax.axis_index("c")
        base = wid * rows_per_worker
        b = lax.div(base, seq_len)
        pos_base = lax.rem(base, seq_len)
        cp_i = pltpu.async_copy(
            ids_hbm.at[b, pl.ds(pos_base, rows_per_worker)], idx_v, sem_i)
        pos_cps = [
            pltpu.async_copy(
                pos_hbm.at[pl.ds(pos_base + j * _CHUNK, _CHUNK)],
                rows_v.at[pl.ds(j * _CHUNK, _CHUNK)],
                sem_p[j],
            )
            for j in range(n_chunks)
        ]
        cp_i.wait()
        gathers = []
        for j in range(n_chunks):
            pos_cps[j].wait()
            gathers.append(pltpu.async_copy(
                table_hbm.at[idx_v.at[pl.ds(j * _CHUNK, _CHUNK)]],
                rows_v.at[pl.ds(j * _CHUNK, _CHUNK)],
                sem_g[j],
                add=True,
            ))
        outs = []
        for j in range(n_chunks):
            gathers[j].wait()
            outs.append(pltpu.async_copy(
                rows_v.at[pl.ds(j * _CHUNK, _CHUNK)],
                out_hbm.at[b, pl.ds(pos_base + j * _CHUNK, _CHUNK)],
                sem_o,
            ))
        for cp in outs:
            cp.wait()

    return emb_kernel


def kernel(input_ids, embedding, position_embedding):
    batch, seq_len = input_ids.shape
    hidden = embedding.shape[1]
    fn = _build(batch, seq_len, hidden)
    return fn(input_ids.astype(jnp.int32), embedding, position_embedding)


# speedup vs baseline: 1.0654x; 1.0296x over previous
import functools

import jax
import jax.numpy as jnp
from jax import lax
from jax.experimental import pallas as pl
from jax.experimental.pallas import tpu as pltpu
from jax.experimental.pallas import tpu_sc as plsc

_LANES = 16


@functools.lru_cache(maxsize=None)
def _build(batch: int, seq_len: int, hidden: int):
    info = plsc.get_sparse_core_info()
    num_workers = info.num_cores * info.num_subcores
    p = seq_len // num_workers
    assert p * num_workers == seq_len
    assert p % 8 == 0 and p <= 128
    assert hidden % _LANES == 0

    mesh = plsc.VectorSubcoreMesh(core_axis_name="c", subcore_axis_name="s")

    @functools.partial(
        pl.kernel,
        out_type=jax.ShapeDtypeStruct((batch, seq_len, hidden), jnp.float32),
        mesh=mesh,
        scratch_types=[
            pltpu.VMEM((batch, p), jnp.int32),
            pltpu.VMEM((batch, p, hidden), jnp.float32),
            pltpu.SemaphoreType.DMA,
            pltpu.SemaphoreType.DMA,
            [pltpu.SemaphoreType.DMA for _ in range(batch)],
            pltpu.SemaphoreType.DMA,
        ],
    )
    def emb_kernel(ids_hbm, table_hbm, pos_hbm, out_hbm,
                   idx_v, rows_v, sem_i, sem_p, sem_g, sem_o):
        wid = lax.axis_index("s") * info.num_cores + lax.axis_index("c")
        lo = wid * p

        idx_cps = [
            pltpu.async_copy(ids_hbm.at[bb, pl.ds(lo, p)], idx_v.at[bb], sem_i)
            for bb in range(batch)
        ]
        cp_p = pltpu.async_copy(pos_hbm.at[pl.ds(lo, p)], rows_v.at[0], sem_p)
        cp_p.wait()

        def replicate(src_b, dst_b):
            def body(r, carry):
                for c in range(hidden // _LANES):
                    sl = pl.ds(c * _LANES, _LANES)
                    rows_v[dst_b, r, sl] = rows_v[src_b, r, sl]
                return carry
            lax.fori_loop(0, p, body, 0)

        gathers = []
        for b in range(batch):
            if b + 1 < batch:
                replicate(b, b + 1)
            if b == 0:
                for cp in idx_cps:
                    cp.wait()
            gathers.append(pltpu.async_copy(
                table_hbm.at[idx_v.at[b]],
                rows_v.at[b],
                sem_g[b],
                add=True,
            ))
        outs = []
        for b in range(batch):
            gathers[b].wait()
            outs.append(pltpu.async_copy(
                rows_v.at[b],
                out_hbm.at[b, pl.ds(lo, p)],
                sem_o,
            ))
        for cp in outs:
            cp.wait()

    return emb_kernel


def kernel(input_ids, embedding, position_embedding):
    batch, seq_len = input_ids.shape
    hidden = embedding.shape[1]
    fn = _build(batch, seq_len, hidden)
    return fn(input_ids.astype(jnp.int32), embedding, position_embedding)
